# baseline (speedup 1.0000x reference)
import functools

import jax
import jax.numpy as jnp
from jax import lax
from jax.experimental import layout as jax_layout
from jax.experimental import pallas as pl
from jax.experimental.pallas import tpu as pltpu
from jax.experimental.pallas import tpu_sc as plsc

NC, NS, L = 2, 16, 16
NW = NC * NS
GRP = 128
CH = 512
N_GRP = CH // GRP
NJ = CH // 128
NBUF = 2
PITCH = 129
PLANE = NJ * 8 + 8


@functools.lru_cache(maxsize=None)
def _make(B, V, D):
  assert B % (NW * CH) == 0 and D == 32
  b_per_w = B // NW
  n_ch = b_per_w // CH
  assert n_ch >= NBUF
  grp_per_w = b_per_w // GRP
  scale = jnp.float32(D) ** 0.5
  mesh = plsc.VectorSubcoreMesh(core_axis_name="c", subcore_axis_name="s")

  @functools.partial(
      pl.kernel,
      out_type=jax.ShapeDtypeStruct((D // 8, B // 128 * 8, 128), jnp.float32),
      mesh=mesh,
      scratch_types=[
          pltpu.VMEM((grp_per_w, GRP), jnp.int32),
          pltpu.VMEM((NBUF, CH, D), jnp.float32),
          pltpu.VMEM((NBUF, 4 * PLANE, PITCH), jnp.float32),
          pltpu.SemaphoreType.DMA((NBUF,)),
          pltpu.SemaphoreType.DMA((NBUF,)),
      ],
      compiler_params=pltpu.CompilerParams(
          use_tc_tiling_on_sc=False,
          needs_layout_passes=False,
      ),
  )
  def k(idx_hbm, table_hbm, out_hbm, idx_v, rows, stag, gsem, wsem):
    wid = lax.axis_index("s") * NC + lax.axis_index("c")
    base = wid * b_per_w
    lanes = lax.iota(jnp.int32, L)
    row_lo = (lanes >> 3) * PLANE + (lanes & 7)
    row_hi = row_lo + 2 * PLANE

    pltpu.sync_copy(
        idx_hbm.at[pl.ds(pl.multiple_of(wid * grp_per_w, 8), grp_per_w)],
        idx_v,
    )

    def start_gather(g, buf):
      for j in range(N_GRP):
        pltpu.async_copy(
            table_hbm.at[idx_v.at[g * N_GRP + j]],
            rows.at[buf].at[pl.ds(j * GRP, GRP)],
            gsem.at[buf],
        )

    def wait_gather(buf):
      for j in range(N_GRP):
        pltpu.make_async_copy(
            table_hbm.at[idx_v.at[j]],
            rows.at[buf].at[pl.ds(j * GRP, GRP)],
            gsem.at[buf],
        ).wait()

    def writeout(g, buf):
      jbase = pl.multiple_of((base + g * CH) // 128 * 8, 8)
      for i in range(4):
        pltpu.async_copy(
            stag.at[buf].at[pl.ds(i * PLANE, NJ * 8), pl.ds(0, 128)],
            out_hbm.at[i].at[pl.ds(jbase, NJ * 8)],
            wsem.at[buf],
        )

    def wait_writeout(buf):
      for i in range(4):
        pltpu.make_async_copy(
            stag.at[buf].at[pl.ds(i * PLANE, NJ * 8), pl.ds(0, 128)],
            out_hbm.at[i].at[pl.ds(0, NJ * 8)],
            wsem.at[buf],
        ).wait()

    start_gather(0, 0)

    @pl.loop(0, n_ch)
    def _chunk(g):
      buf = lax.rem(g, NBUF)
      wait_gather(buf)

      @pl.when(g + 1 < n_ch)
      def _():
        start_gather(g + 1, lax.rem(g + 1, NBUF))

      @pl.when(g >= NBUF)
      def _():
        wait_writeout(buf)

      @plsc.parallel_loop(0, CH, unroll=8)
      def _transpose_scale(r):
        roff = lax.shift_right_logical(r, 7) * 8
        col = jnp.full((L,), r & 127, jnp.int32)
        plsc.store_scatter(
            stag.at[buf],
            [row_lo + roff, col],
            rows[buf, r, pl.ds(0, L)] * scale,
        )
        plsc.store_scatter(
            stag.at[buf],
            [row_hi + roff, col],
            rows[buf, r, pl.ds(L, L)] * scale,
        )

      writeout(g, buf)

    for c in range(n_ch - NBUF, n_ch):
      wait_writeout(c % NBUF)

  return k


def _row_major_tiled_format():
  try:
    mesh = jax.sharding.get_mesh()
  except ValueError:
    mesh = None
  if mesh is not None and not mesh.empty:
    dev = mesh.devices.flat[0]
  else:
    dev = jax.devices()[0]
  del dev
  return jax_layout.Layout((0, 1))


_TABLE_FMT = _row_major_tiled_format()


def _impl(inputs, lookup_table):
  B0, B1 = inputs.shape
  V, D = lookup_table.shape
  B = B0 * B1
  idx = inputs.reshape(B // GRP, GRP).astype(jnp.int32)
  t128 = jax_layout.with_layout_constraint(
      lookup_table.reshape(V // 4, 128), _TABLE_FMT
  )
  table_lin = t128.reshape(V, D)
  out3 = _make(B, V, D)(idx, table_lin)
  out4 = out3.reshape(4, B // 128, 8, 128)
  out2 = out4.transpose(1, 3, 0, 2).reshape(B, D)
  return out2.reshape(B0, B1, D)


kernel = jax.jit(_impl)


# speedup vs baseline: 1.0006x; 1.0006x over previous
import functools

import jax
import jax.numpy as jnp
from jax import lax
from jax.experimental import pallas as pl
from jax.experimental.pallas import tpu as pltpu
from jax.experimental.pallas import tpu_sc as plsc

NC, NS, L = 2, 16, 16
NW = NC * NS
GRP = 128
CH = 512
N_GRP = CH // GRP
NJ = CH // 128
NBUF = 2
PITCH = 129
PLANE = NJ * 8 + 8


@functools.lru_cache(maxsize=None)
def _make(B, V, D):
  assert B % (NW * CH) == 0 and D == 32
  b_per_w = B // NW
  n_ch = b_per_w // CH
  assert n_ch >= NBUF
  grp_per_w = b_per_w // GRP
  scale = jnp.float32(D) ** 0.5
  mesh = plsc.VectorSubcoreMesh(core_axis_name="c", subcore_axis_name="s")

  @functools.partial(
      pl.kernel,
      out_type=jax.ShapeDtypeStruct((D // 8, B // 128 * 8, 128), jnp.float32),
      mesh=mesh,
      scratch_types=[
          pltpu.VMEM((grp_per_w, GRP), jnp.int32),
          pltpu.VMEM((NBUF, CH, D), jnp.float32),
          pltpu.VMEM((NBUF, 4 * PLANE, PITCH), jnp.float32),
          pltpu.SemaphoreType.DMA((NBUF,)),
          pltpu.SemaphoreType.DMA((NBUF,)),
      ],
      compiler_params=pltpu.CompilerParams(
          use_tc_tiling_on_sc=False,
          needs_layout_passes=False,
      ),
  )
  def k(idx_hbm, table_hbm, out_hbm, idx_v, rows, stag, gsem, wsem):
    wid = lax.axis_index("s") * NC + lax.axis_index("c")
    base = wid * b_per_w
    lanes = lax.iota(jnp.int32, L)
    row_lo = (lanes >> 3) * PLANE + (lanes & 7)
    row_hi = row_lo + 2 * PLANE

    pltpu.sync_copy(
        idx_hbm.at[pl.ds(pl.multiple_of(wid * grp_per_w, 8), grp_per_w)],
        idx_v,
    )

    def start_gather(g, buf):
      for j in range(N_GRP):
        pltpu.async_copy(
            table_hbm.at[idx_v.at[g * N_GRP + j]],
            rows.at[buf].at[pl.ds(j * GRP, GRP)],
            gsem.at[buf],
        )

    def wait_gather(buf):
      for j in range(N_GRP):
        pltpu.make_async_copy(
            table_hbm.at[idx_v.at[j]],
            rows.at[buf].at[pl.ds(j * GRP, GRP)],
            gsem.at[buf],
        ).wait()

    def writeout(g, buf):
      jbase = pl.multiple_of((base + g * CH) // 128 * 8, 8)
      for i in range(4):
        pltpu.async_copy(
            stag.at[buf].at[pl.ds(i * PLANE, NJ * 8), pl.ds(0, 128)],
            out_hbm.at[i].at[pl.ds(jbase, NJ * 8)],
            wsem.at[buf],
        )

    def wait_writeout(buf):
      for i in range(4):
        pltpu.make_async_copy(
            stag.at[buf].at[pl.ds(i * PLANE, NJ * 8), pl.ds(0, 128)],
            out_hbm.at[i].at[pl.ds(0, NJ * 8)],
            wsem.at[buf],
        ).wait()

    start_gather(0, 0)

    @pl.loop(0, n_ch)
    def _chunk(g):
      buf = lax.rem(g, NBUF)
      wait_gather(buf)

      @pl.when(g + 1 < n_ch)
      def _():
        start_gather(g + 1, lax.rem(g + 1, NBUF))

      @pl.when(g >= NBUF)
      def _():
        wait_writeout(buf)

      @plsc.parallel_loop(0, CH, unroll=8)
      def _transpose_scale(r):
        roff = lax.shift_right_logical(r, 7) * 8
        col = jnp.full((L,), r & 127, jnp.int32)
        plsc.store_scatter(
            stag.at[buf],
            [row_lo + roff, col],
            rows[buf, r, pl.ds(0, L)] * scale,
        )
        plsc.store_scatter(
            stag.at[buf],
            [row_hi + roff, col],
            rows[buf, r, pl.ds(L, L)] * scale,
        )

      writeout(g, buf)

    for c in range(n_ch - NBUF, n_ch):
      wait_writeout(c % NBUF)

  return k


def _impl(inputs, lookup_table):
  B0, B1 = inputs.shape
  V, D = lookup_table.shape
  B = B0 * B1
  idx = inputs.reshape(B // GRP, GRP).astype(jnp.int32)
  out3 = _make(B, V, D)(idx, lookup_table)
  out4 = out3.reshape(4, B // 128, 8, 128)
  out2 = out4.transpose(1, 3, 0, 2).reshape(B, D)
  return out2.reshape(B0, B1, D)


kernel = jax.jit(_impl)
